# baseline (speedup 1.0000x reference)
import functools

import jax
import jax.numpy as jnp
from jax import lax
from jax.experimental import pallas as pl
from jax.experimental.pallas import tpu as pltpu
from jax.experimental.pallas import tpu_sc as plsc

N_USERS = 3000
N_ITEMS = 7000
N_NODES = N_USERS + N_ITEMS
EMBED = 128
N_EDGES = 320000
LAYERS = 3

NC = 2
NS = 16
NW = NC * NS
K = 128
NCHUNK = -(-N_EDGES // (NW * K))
EPT = NCHUNK * K
N_EDGES_PAD = EPT * NW
STRIPE = 624
STRIPE_LAST = N_NODES - STRIPE * (NS - 1)
LANES = 16
EG = EMBED // LANES

_mesh = plsc.VectorSubcoreMesh(core_axis_name="c", subcore_axis_name="s")


NBUF = 3
NGRP = (NCHUNK - 2) // NBUF


@functools.partial(
    pl.kernel,
    out_type=[
        jax.ShapeDtypeStruct((N_NODES, EMBED), jnp.float32),
        jax.ShapeDtypeStruct((N_NODES, EMBED), jnp.float32),
    ],
    mesh=_mesh,
    scratch_types=[
        pltpu.VMEM_SHARED((N_NODES, EMBED), jnp.float32),
        pltpu.VMEM((K,), jnp.int32), pltpu.VMEM((K,), jnp.int32),
        pltpu.VMEM((K,), jnp.int32),
        pltpu.VMEM((K,), jnp.int32), pltpu.VMEM((K,), jnp.int32),
        pltpu.VMEM((K,), jnp.int32),
        pltpu.VMEM((K,), jnp.float32), pltpu.VMEM((K,), jnp.float32),
        pltpu.VMEM((K,), jnp.float32),
        pltpu.VMEM((K, EMBED), jnp.float32), pltpu.VMEM((K, EMBED), jnp.float32),
        pltpu.VMEM((K, EMBED), jnp.float32),
        pltpu.SemaphoreType.DMA, pltpu.SemaphoreType.DMA,
        pltpu.SemaphoreType.DMA,
        pltpu.SemaphoreType.DMA, pltpu.SemaphoreType.DMA,
        pltpu.SemaphoreType.DMA,
        pltpu.SemaphoreType.DMA, pltpu.SemaphoreType.DMA,
        pltpu.SemaphoreType.DMA,
    ],
)
def _propagate_kernel(x_hbm, src_hbm, dst_hbm, w_hbm, zeros_hbm,
                      p0_hbm, p1_hbm, acc,
                      src0, src1, src2, dst0, dst1, dst2, w0, w1, w2,
                      msg0, msg1, msg2,
                      isem0, isem1, isem2, gsem0, gsem1, gsem2,
                      ssem0, ssem1, ssem2):
    c = lax.axis_index("c")
    s = lax.axis_index("s")
    wid = s * NC + c

    srcs = (src0, src1, src2)
    dsts = (dst0, dst1, dst2)
    ws = (w0, w1, w2)
    msgs = (msg0, msg1, msg2)
    isems = (isem0, isem1, isem2)
    gsems = (gsem0, gsem1, gsem2)
    ssems = (ssem0, ssem1, ssem2)

    @pl.when(s < NS - 1)
    def _():
        pltpu.sync_copy(zeros_hbm.at[pl.ds(0, STRIPE)],
                        acc.at[pl.ds(s * STRIPE, STRIPE)])

    @pl.when(s == NS - 1)
    def _():
        pltpu.sync_copy(zeros_hbm, acc.at[pl.ds(s * STRIPE, STRIPE_LAST)])

    plsc.subcore_barrier()

    base = wid * EPT

    def fire_idx(j, b):
        off = base + j * K
        pltpu.async_copy(src_hbm.at[pl.ds(off, K)], srcs[b], isems[b])
        pltpu.async_copy(dst_hbm.at[pl.ds(off, K)], dsts[b], isems[b])
        pltpu.async_copy(w_hbm.at[pl.ds(off, K)], ws[b], isems[b])

    def drain_idx(b):
        pltpu.make_async_copy(src_hbm.at[pl.ds(0, K)], srcs[b], isems[b]).wait()
        pltpu.make_async_copy(dst_hbm.at[pl.ds(0, K)], dsts[b], isems[b]).wait()
        pltpu.make_async_copy(w_hbm.at[pl.ds(0, K)], ws[b], isems[b]).wait()

    def fire_gather(b):
        pltpu.async_copy(x_hbm.at[srcs[b]], msgs[b], gsems[b])

    def drain_gather(b):
        pltpu.make_async_copy(x_hbm.at[srcs[b]], msgs[b], gsems[b]).wait()

    def fire_scatter(b):
        pltpu.async_copy(msgs[b], acc.at[dsts[b]], ssems[b], add=True)

    def drain_scatter(b):
        pltpu.make_async_copy(msgs[b], acc.at[dsts[b]], ssems[b]).wait()

    def scale(b):
        def grp_body(gi, carry):
            e0 = gi * LANES
            w16 = ws[b][pl.ds(e0, LANES)]
            for l in range(LANES):
                wb = w16[l]
                for g in range(EG):
                    sl = pl.ds(g * LANES, LANES)
                    msgs[b][e0 + l, sl] = msgs[b][e0 + l, sl] * wb
            return carry

        lax.fori_loop(0, K // LANES, grp_body, 0)

    fire_idx(0, 0)
    fire_idx(1, 1)
    drain_idx(0)
    fire_gather(0)

    def group_body(jg, carry):
        j0 = jg * NBUF
        for b in range(NBUF):
            j = j0 + b
            b1 = (b + 1) % NBUF
            b2 = (b + 2) % NBUF
            drain_gather(b)
            scale(b)
            fire_scatter(b)
            drain_idx(b1)
            fire_gather(b1)
            if b == 0:
                @pl.when(jg > 0)
                def _():
                    drain_scatter(b2)
            else:
                drain_scatter(b2)
            fire_idx(j + 2, b2)
        return carry

    lax.fori_loop(0, NGRP, group_body, 0)

    for j in range(NGRP * NBUF, NCHUNK):
        b = j % NBUF
        b1 = (b + 1) % NBUF
        b2 = (b + 2) % NBUF
        drain_gather(b)
        scale(b)
        fire_scatter(b)
        if j + 1 < NCHUNK:
            drain_idx(b1)
            fire_gather(b1)
        drain_scatter(b2)
        if j + 2 < NCHUNK:
            fire_idx(j + 2, b2)
    drain_scatter((NCHUNK - 1) % NBUF)

    plsc.subcore_barrier()

    for last in (False, True):
        n = STRIPE_LAST if last else STRIPE
        cond = (s == NS - 1) if last else (s < NS - 1)

        @pl.when((c == 0) & cond)
        def _(n=n):
            stripe = pl.ds(s * STRIPE, n)
            pltpu.sync_copy(acc.at[stripe], p0_hbm.at[stripe])

        @pl.when((c == 1) & cond)
        def _(n=n):
            stripe = pl.ds(s * STRIPE, n)
            pltpu.sync_copy(acc.at[stripe], p1_hbm.at[stripe])


CB = 2000


def _make_combine_tc(scale, emit_x):
    bs = pl.BlockSpec((CB, EMBED), lambda i: (i, 0))
    out_shape = [jax.ShapeDtypeStruct((N_NODES, EMBED), jnp.float32)]
    if emit_x:
        out_shape = out_shape * 2

    def body(p0_ref, p1_ref, sum_ref, *out_refs):
        xv = p0_ref[...] + p1_ref[...]
        if emit_x:
            out_refs[0][...] = xv
            out_refs[1][...] = (sum_ref[...] + xv) * scale
        else:
            out_refs[0][...] = (sum_ref[...] + xv) * scale

    return pl.pallas_call(
        body,
        grid=(N_NODES // CB,),
        in_specs=[bs, bs, bs],
        out_specs=[bs] * len(out_shape),
        out_shape=out_shape,
    )


_combine_mid = _make_combine_tc(1.0, emit_x=True)
_combine_last = _make_combine_tc(1.0 / (LAYERS + 1), emit_x=False)


def kernel(user_table, item_table, edge_index, edge_weight):
    x = jnp.concatenate([user_table, item_table], axis=0)
    src = edge_index[0].astype(jnp.int32)
    dst = edge_index[1].astype(jnp.int32)
    w = edge_weight.astype(jnp.float32)
    pad = N_EDGES_PAD - N_EDGES
    src = jnp.concatenate([src, jnp.zeros((pad,), jnp.int32)])
    dst = jnp.concatenate([dst, jnp.zeros((pad,), jnp.int32)])
    w = jnp.concatenate([w, jnp.zeros((pad,), jnp.float32)])
    zeros = jnp.zeros((STRIPE_LAST, EMBED), jnp.float32)

    run_sum = x
    for layer in range(LAYERS):
        p0, p1 = _propagate_kernel(x, src, dst, w, zeros)
        if layer < LAYERS - 1:
            x, run_sum = _combine_mid(p0, p1, run_sum)
        else:
            (f,) = _combine_last(p0, p1, run_sum)

    return f[:N_USERS], f[N_USERS:]


# speedup vs baseline: 2.5790x; 2.5790x over previous
import functools

import jax
import jax.numpy as jnp
from jax import lax
from jax.experimental import pallas as pl
from jax.experimental.pallas import tpu as pltpu
from jax.experimental.pallas import tpu_sc as plsc

N_USERS = 3000
N_ITEMS = 7000
N_NODES = N_USERS + N_ITEMS
EMBED = 128
N_EDGES = 320000
LAYERS = 3

NC = 2
NS = 16
NW = NC * NS
K = 128
NCHUNK = -(-N_EDGES // (NW * K))
EPT = NCHUNK * K
N_EDGES_PAD = EPT * NW
STRIPE = 624
STRIPE_LAST = N_NODES - STRIPE * (NS - 1)
LANES = 16
EG = EMBED // LANES

_mesh = plsc.VectorSubcoreMesh(core_axis_name="c", subcore_axis_name="s")


NBUF = 3
NGRP = (NCHUNK - 2) // NBUF


@functools.partial(
    pl.kernel,
    out_type=[
        jax.ShapeDtypeStruct((N_NODES, EMBED), jnp.float32),
        jax.ShapeDtypeStruct((N_NODES, EMBED), jnp.float32),
    ],
    mesh=_mesh,
    scratch_types=[
        pltpu.VMEM_SHARED((N_NODES, EMBED), jnp.float32),
        pltpu.VMEM((K,), jnp.int32), pltpu.VMEM((K,), jnp.int32),
        pltpu.VMEM((K,), jnp.int32),
        pltpu.VMEM((K,), jnp.int32), pltpu.VMEM((K,), jnp.int32),
        pltpu.VMEM((K,), jnp.int32),
        pltpu.VMEM((K,), jnp.float32), pltpu.VMEM((K,), jnp.float32),
        pltpu.VMEM((K,), jnp.float32),
        pltpu.VMEM((K, EMBED), jnp.float32), pltpu.VMEM((K, EMBED), jnp.float32),
        pltpu.VMEM((K, EMBED), jnp.float32),
        pltpu.SemaphoreType.DMA, pltpu.SemaphoreType.DMA,
        pltpu.SemaphoreType.DMA,
        pltpu.SemaphoreType.DMA, pltpu.SemaphoreType.DMA,
        pltpu.SemaphoreType.DMA,
        pltpu.SemaphoreType.DMA, pltpu.SemaphoreType.DMA,
        pltpu.SemaphoreType.DMA,
    ],
)
def _propagate_kernel(x_hbm, src_hbm, dst_hbm, w_hbm, zeros_hbm,
                      p0_hbm, p1_hbm, acc,
                      src0, src1, src2, dst0, dst1, dst2, w0, w1, w2,
                      msg0, msg1, msg2,
                      isem0, isem1, isem2, gsem0, gsem1, gsem2,
                      ssem0, ssem1, ssem2):
    c = lax.axis_index("c")
    s = lax.axis_index("s")
    wid = s * NC + c

    srcs = (src0, src1, src2)
    dsts = (dst0, dst1, dst2)
    ws = (w0, w1, w2)
    msgs = (msg0, msg1, msg2)
    isems = (isem0, isem1, isem2)
    gsems = (gsem0, gsem1, gsem2)
    ssems = (ssem0, ssem1, ssem2)

    @pl.when(s < NS - 1)
    def _():
        pltpu.sync_copy(zeros_hbm.at[pl.ds(0, STRIPE)],
                        acc.at[pl.ds(s * STRIPE, STRIPE)])

    @pl.when(s == NS - 1)
    def _():
        pltpu.sync_copy(zeros_hbm, acc.at[pl.ds(s * STRIPE, STRIPE_LAST)])

    plsc.subcore_barrier()

    base = wid * EPT

    def fire_idx(j, b):
        off = base + j * K
        pltpu.async_copy(src_hbm.at[pl.ds(off, K)], srcs[b], isems[b])
        pltpu.async_copy(dst_hbm.at[pl.ds(off, K)], dsts[b], isems[b])
        pltpu.async_copy(w_hbm.at[pl.ds(off, K)], ws[b], isems[b])

    def drain_idx(b):
        pltpu.make_async_copy(src_hbm.at[pl.ds(0, K)], srcs[b], isems[b]).wait()
        pltpu.make_async_copy(dst_hbm.at[pl.ds(0, K)], dsts[b], isems[b]).wait()
        pltpu.make_async_copy(w_hbm.at[pl.ds(0, K)], ws[b], isems[b]).wait()

    def fire_gather(b):
        pltpu.async_copy(x_hbm.at[srcs[b]], msgs[b], gsems[b])

    def drain_gather(b):
        pltpu.make_async_copy(x_hbm.at[srcs[b]], msgs[b], gsems[b]).wait()

    def fire_scatter(b):
        pltpu.async_copy(msgs[b], acc.at[dsts[b]], ssems[b], add=True)

    def drain_scatter(b):
        pltpu.make_async_copy(msgs[b], acc.at[dsts[b]], ssems[b]).wait()

    def scale(b):
        def grp_body(gi, carry):
            e0 = gi * LANES
            w16 = ws[b][pl.ds(e0, LANES)]
            for l in range(LANES):
                wb = w16[l]
                for g in range(EG):
                    sl = pl.ds(g * LANES, LANES)
                    msgs[b][e0 + l, sl] = msgs[b][e0 + l, sl] * wb
            return carry

        lax.fori_loop(0, K // LANES, grp_body, 0)

    fire_idx(0, 0)
    fire_idx(1, 1)
    drain_idx(0)
    fire_gather(0)

    def group_body(jg, carry):
        j0 = jg * NBUF
        for b in range(NBUF):
            j = j0 + b
            b1 = (b + 1) % NBUF
            b2 = (b + 2) % NBUF
            drain_idx(b1)
            fire_gather(b1)
            drain_gather(b)
            scale(b)
            fire_scatter(b)
            if b == 0:
                @pl.when(jg > 0)
                def _():
                    drain_scatter(b2)
            else:
                drain_scatter(b2)
            fire_idx(j + 2, b2)
        return carry

    lax.fori_loop(0, NGRP, group_body, 0)

    for j in range(NGRP * NBUF, NCHUNK):
        b = j % NBUF
        b1 = (b + 1) % NBUF
        b2 = (b + 2) % NBUF
        if j + 1 < NCHUNK:
            drain_idx(b1)
            fire_gather(b1)
        drain_gather(b)
        scale(b)
        fire_scatter(b)
        drain_scatter(b2)
        if j + 2 < NCHUNK:
            fire_idx(j + 2, b2)
    drain_scatter((NCHUNK - 1) % NBUF)

    plsc.subcore_barrier()

    for last in (False, True):
        n = STRIPE_LAST if last else STRIPE
        cond = (s == NS - 1) if last else (s < NS - 1)

        @pl.when((c == 0) & cond)
        def _(n=n):
            stripe = pl.ds(s * STRIPE, n)
            pltpu.sync_copy(acc.at[stripe], p0_hbm.at[stripe])

        @pl.when((c == 1) & cond)
        def _(n=n):
            stripe = pl.ds(s * STRIPE, n)
            pltpu.sync_copy(acc.at[stripe], p1_hbm.at[stripe])


CB = 2000


def _make_combine_tc(scale, emit_x):
    bs = pl.BlockSpec((CB, EMBED), lambda i: (i, 0))
    out_shape = [jax.ShapeDtypeStruct((N_NODES, EMBED), jnp.float32)]
    if emit_x:
        out_shape = out_shape * 2

    def body(p0_ref, p1_ref, sum_ref, *out_refs):
        xv = p0_ref[...] + p1_ref[...]
        if emit_x:
            out_refs[0][...] = xv
            out_refs[1][...] = (sum_ref[...] + xv) * scale
        else:
            out_refs[0][...] = (sum_ref[...] + xv) * scale

    return pl.pallas_call(
        body,
        grid=(N_NODES // CB,),
        in_specs=[bs, bs, bs],
        out_specs=[bs] * len(out_shape),
        out_shape=out_shape,
    )


_combine_mid = _make_combine_tc(1.0, emit_x=True)
_combine_last = _make_combine_tc(1.0 / (LAYERS + 1), emit_x=False)


def kernel(user_table, item_table, edge_index, edge_weight):
    x = jnp.concatenate([user_table, item_table], axis=0)
    src = edge_index[0].astype(jnp.int32)
    dst = edge_index[1].astype(jnp.int32)
    w = edge_weight.astype(jnp.float32)
    pad = N_EDGES_PAD - N_EDGES
    spread = (jnp.arange(pad, dtype=jnp.int32) * 8) % N_NODES
    src = jnp.concatenate([src, spread])
    dst = jnp.concatenate([dst, spread])
    w = jnp.concatenate([w, jnp.zeros((pad,), jnp.float32)])
    zeros = jnp.zeros((STRIPE_LAST, EMBED), jnp.float32)

    run_sum = x
    for layer in range(LAYERS):
        p0, p1 = _propagate_kernel(x, src, dst, w, zeros)
        if layer < LAYERS - 1:
            x, run_sum = _combine_mid(p0, p1, run_sum)
        else:
            (f,) = _combine_last(p0, p1, run_sum)

    return f[:N_USERS], f[N_USERS:]


# speedup vs baseline: 2.5883x; 1.0036x over previous
import functools

import jax
import jax.numpy as jnp
from jax import lax
from jax.experimental import pallas as pl
from jax.experimental.pallas import tpu as pltpu
from jax.experimental.pallas import tpu_sc as plsc

N_USERS = 3000
N_ITEMS = 7000
N_NODES = N_USERS + N_ITEMS
EMBED = 128
N_EDGES = 320000
LAYERS = 3

NC = 2
NS = 16
NW = NC * NS
K = 128
NCHUNK = -(-N_EDGES // (NW * K))
EPT = NCHUNK * K
N_EDGES_PAD = EPT * NW
STRIPE = 624
STRIPE_LAST = N_NODES - STRIPE * (NS - 1)
LANES = 16
EG = EMBED // LANES

_mesh = plsc.VectorSubcoreMesh(core_axis_name="c", subcore_axis_name="s")


NBUF = 3
NGRP = (NCHUNK - 2) // NBUF


@functools.partial(
    pl.kernel,
    out_type=[
        jax.ShapeDtypeStruct((N_NODES, EMBED), jnp.float32),
        jax.ShapeDtypeStruct((N_NODES, EMBED), jnp.float32),
    ],
    mesh=_mesh,
    scratch_types=[
        pltpu.VMEM_SHARED((N_NODES, EMBED), jnp.float32),
        pltpu.VMEM((K,), jnp.int32), pltpu.VMEM((K,), jnp.int32),
        pltpu.VMEM((K,), jnp.int32),
        pltpu.VMEM((K,), jnp.int32), pltpu.VMEM((K,), jnp.int32),
        pltpu.VMEM((K,), jnp.int32),
        pltpu.VMEM((K,), jnp.float32), pltpu.VMEM((K,), jnp.float32),
        pltpu.VMEM((K,), jnp.float32),
        pltpu.VMEM((K, EMBED), jnp.float32), pltpu.VMEM((K, EMBED), jnp.float32),
        pltpu.VMEM((K, EMBED), jnp.float32),
        pltpu.SemaphoreType.DMA, pltpu.SemaphoreType.DMA,
        pltpu.SemaphoreType.DMA,
        pltpu.SemaphoreType.DMA, pltpu.SemaphoreType.DMA,
        pltpu.SemaphoreType.DMA,
        pltpu.SemaphoreType.DMA, pltpu.SemaphoreType.DMA,
        pltpu.SemaphoreType.DMA,
    ],
)
def _propagate_kernel(x_hbm, src_hbm, dst_hbm, w_hbm, zeros_hbm,
                      p0_hbm, p1_hbm, acc,
                      src0, src1, src2, dst0, dst1, dst2, w0, w1, w2,
                      msg0, msg1, msg2,
                      isem0, isem1, isem2, gsem0, gsem1, gsem2,
                      ssem0, ssem1, ssem2):
    c = lax.axis_index("c")
    s = lax.axis_index("s")
    wid = s * NC + c

    srcs = (src0, src1, src2)
    dsts = (dst0, dst1, dst2)
    ws = (w0, w1, w2)
    msgs = (msg0, msg1, msg2)
    isems = (isem0, isem1, isem2)
    gsems = (gsem0, gsem1, gsem2)
    ssems = (ssem0, ssem1, ssem2)

    base = wid * EPT

    def fire_idx(j, b):
        off = base + j * K
        pltpu.async_copy(src_hbm.at[pl.ds(off, K)], srcs[b], isems[b])
        pltpu.async_copy(dst_hbm.at[pl.ds(off, K)], dsts[b], isems[b])
        pltpu.async_copy(w_hbm.at[pl.ds(off, K)], ws[b], isems[b])

    def drain_idx(b):
        pltpu.make_async_copy(src_hbm.at[pl.ds(0, K)], srcs[b], isems[b]).wait()
        pltpu.make_async_copy(dst_hbm.at[pl.ds(0, K)], dsts[b], isems[b]).wait()
        pltpu.make_async_copy(w_hbm.at[pl.ds(0, K)], ws[b], isems[b]).wait()

    def fire_gather(b):
        pltpu.async_copy(x_hbm.at[srcs[b]], msgs[b], gsems[b])

    def drain_gather(b):
        pltpu.make_async_copy(x_hbm.at[srcs[b]], msgs[b], gsems[b]).wait()

    def fire_scatter(b):
        pltpu.async_copy(msgs[b], acc.at[dsts[b]], ssems[b], add=True)

    def drain_scatter(b):
        pltpu.make_async_copy(msgs[b], acc.at[dsts[b]], ssems[b]).wait()

    def scale(b):
        def grp_body(gi, carry):
            e0 = gi * LANES
            w16 = ws[b][pl.ds(e0, LANES)]
            for l in range(LANES):
                wb = w16[l]
                for g in range(EG):
                    sl = pl.ds(g * LANES, LANES)
                    msgs[b][e0 + l, sl] = msgs[b][e0 + l, sl] * wb
            return carry

        lax.fori_loop(0, K // LANES, grp_body, 0)

    fire_idx(0, 0)
    fire_idx(1, 1)
    drain_idx(0)
    fire_gather(0)

    @pl.when(s < NS - 1)
    def _():
        pltpu.sync_copy(zeros_hbm.at[pl.ds(0, STRIPE)],
                        acc.at[pl.ds(s * STRIPE, STRIPE)])

    @pl.when(s == NS - 1)
    def _():
        pltpu.sync_copy(zeros_hbm, acc.at[pl.ds(s * STRIPE, STRIPE_LAST)])

    plsc.subcore_barrier()

    def group_body(jg, carry):
        j0 = jg * NBUF
        for b in range(NBUF):
            j = j0 + b
            b1 = (b + 1) % NBUF
            b2 = (b + 2) % NBUF
            drain_idx(b1)
            fire_gather(b1)
            drain_gather(b)
            scale(b)
            fire_scatter(b)
            if b == 0:
                @pl.when(jg > 0)
                def _():
                    drain_scatter(b2)
            else:
                drain_scatter(b2)
            fire_idx(j + 2, b2)
        return carry

    lax.fori_loop(0, NGRP, group_body, 0)

    for j in range(NGRP * NBUF, NCHUNK):
        b = j % NBUF
        b1 = (b + 1) % NBUF
        b2 = (b + 2) % NBUF
        if j + 1 < NCHUNK:
            drain_idx(b1)
            fire_gather(b1)
        drain_gather(b)
        scale(b)
        fire_scatter(b)
        drain_scatter(b2)
        if j + 2 < NCHUNK:
            fire_idx(j + 2, b2)
    drain_scatter((NCHUNK - 1) % NBUF)

    plsc.subcore_barrier()

    for last in (False, True):
        n = STRIPE_LAST if last else STRIPE
        cond = (s == NS - 1) if last else (s < NS - 1)

        @pl.when((c == 0) & cond)
        def _(n=n):
            stripe = pl.ds(s * STRIPE, n)
            pltpu.sync_copy(acc.at[stripe], p0_hbm.at[stripe])

        @pl.when((c == 1) & cond)
        def _(n=n):
            stripe = pl.ds(s * STRIPE, n)
            pltpu.sync_copy(acc.at[stripe], p1_hbm.at[stripe])


CB = 2000


def _make_combine_tc(scale, emit_x):
    bs = pl.BlockSpec((CB, EMBED), lambda i: (i, 0))
    out_shape = [jax.ShapeDtypeStruct((N_NODES, EMBED), jnp.float32)]
    if emit_x:
        out_shape = out_shape * 2

    def body(p0_ref, p1_ref, sum_ref, *out_refs):
        xv = p0_ref[...] + p1_ref[...]
        if emit_x:
            out_refs[0][...] = xv
            out_refs[1][...] = (sum_ref[...] + xv) * scale
        else:
            out_refs[0][...] = (sum_ref[...] + xv) * scale

    return pl.pallas_call(
        body,
        grid=(N_NODES // CB,),
        in_specs=[bs, bs, bs],
        out_specs=[bs] * len(out_shape),
        out_shape=out_shape,
    )


_combine_mid = _make_combine_tc(1.0, emit_x=True)
_combine_last = _make_combine_tc(1.0 / (LAYERS + 1), emit_x=False)


def kernel(user_table, item_table, edge_index, edge_weight):
    x = jnp.concatenate([user_table, item_table], axis=0)
    src = edge_index[0].astype(jnp.int32)
    dst = edge_index[1].astype(jnp.int32)
    w = edge_weight.astype(jnp.float32)
    pad = N_EDGES_PAD - N_EDGES
    spread = (jnp.arange(pad, dtype=jnp.int32) * 8) % N_NODES
    src = jnp.concatenate([src, spread])
    dst = jnp.concatenate([dst, spread])
    w = jnp.concatenate([w, jnp.zeros((pad,), jnp.float32)])
    zeros = jnp.zeros((STRIPE_LAST, EMBED), jnp.float32)

    run_sum = x
    for layer in range(LAYERS):
        p0, p1 = _propagate_kernel(x, src, dst, w, zeros)
        if layer < LAYERS - 1:
            x, run_sum = _combine_mid(p0, p1, run_sum)
        else:
            (f,) = _combine_last(p0, p1, run_sum)

    return f[:N_USERS], f[N_USERS:]
